# Initial kernel scaffold; baseline (speedup 1.0000x reference)
import jax
import jax.numpy as jnp
from jax.experimental import pallas as pl


def kernel(node_emb, segment_ids, weight, bias, scale):
    raise NotImplementedError("write your pallas kernel here")


# speedup vs baseline: 1.7475x; 1.7475x over previous
import functools

import jax
import jax.numpy as jnp
from jax import lax
from jax.experimental import pallas as pl
from jax.experimental.pallas import tpu as pltpu
from jax.experimental.pallas import tpu_sc as plsc

N = 100000
D = 128
G = 256
L = 16
NC = D // L
NW = 32

TOT_HEX = N // 16
BASE_HEX = TOT_HEX // NW
EXTRA = TOT_HEX - BASE_HEX * NW

MAX_ROWS = (BASE_HEX + 1) * 16


def _wid():
    return lax.axis_index("s") * 2 + lax.axis_index("c")


def _start_row(wid):
    return (wid * BASE_HEX + jnp.minimum(wid, EXTRA)) * 16


def _stats_body(x_hbm, ids_hbm, psum_hbm, psq_hbm, phist_hbm,
                ids_v, xbuf, acc, accsq, hist):
    wid = _wid()
    start_row = _start_row(wid)

    zero16 = jnp.zeros((L,), jnp.float32)

    def zbody(g, c_):
        for c in range(NC):
            acc[g, pl.ds(c * L, L)] = zero16
            accsq[g, pl.ds(c * L, L)] = zero16
        hist[g, :] = zero16
        return c_

    lax.fori_loop(0, G, zbody, 0)

    def run(n_hex):
        pltpu.sync_copy(ids_hbm.at[pl.ds(start_row, n_hex * 16)],
                        ids_v.at[pl.ds(0, n_hex * 16)])

        def hex_body(i, c_):
            pltpu.sync_copy(x_hbm.at[pl.ds(start_row + i * 16, 16), :], xbuf)
            idv = ids_v[pl.ds(i * 16, 16)]
            for r in range(16):
                s = idv[r]
                for c in range(NC):
                    sl = pl.ds(c * L, L)
                    xv = xbuf[r, sl]
                    acc[s, sl] += xv
                    accsq[s, sl] += xv * xv
                hist[s, :] += 1.0
            return c_

        lax.fori_loop(0, n_hex, hex_body, 0)

    pl.when(wid < EXTRA)(lambda: run(BASE_HEX + 1))
    pl.when(wid >= EXTRA)(lambda: run(BASE_HEX))

    pltpu.sync_copy(acc, psum_hbm.at[wid])
    pltpu.sync_copy(accsq, psq_hbm.at[wid])
    pltpu.sync_copy(hist, phist_hbm.at[wid])


def _combine_body(psum_ref, psq_ref, phist_ref, w_ref, b_ref, s_ref,
                  a_ref, c_ref):
    sums = jnp.sum(psum_ref[...], axis=0)
    sqs = jnp.sum(psq_ref[...], axis=0)
    cnt = jnp.sum(phist_ref[...], axis=0)[:, :1]
    cnt = jnp.maximum(cnt, 1.0)
    mean = sums / cnt
    t = mean * s_ref[...]
    var_sum = sqs - 2.0 * t * sums + cnt * t * t
    var = jnp.maximum(var_sum, 0.0) / cnt
    rstd = lax.rsqrt(var + 1e-8)
    a = w_ref[...] * rstd
    a_ref[...] = a
    c_ref[...] = b_ref[...] - a * t


def _norm_body(x_hbm, ids_hbm, a_hbm, c_hbm, out_hbm,
               ids_v, xbuf, obuf, av, cv):
    wid = _wid()
    start_row = _start_row(wid)

    pltpu.sync_copy(a_hbm, av)
    pltpu.sync_copy(c_hbm, cv)

    def run(n_hex):
        pltpu.sync_copy(ids_hbm.at[pl.ds(start_row, n_hex * 16)],
                        ids_v.at[pl.ds(0, n_hex * 16)])

        def hex_body(i, c_):
            pltpu.sync_copy(x_hbm.at[pl.ds(start_row + i * 16, 16), :], xbuf)
            idv = ids_v[pl.ds(i * 16, 16)]
            for r in range(16):
                s = idv[r]
                for c in range(NC):
                    sl = pl.ds(c * L, L)
                    obuf[r, sl] = xbuf[r, sl] * av[s, sl] + cv[s, sl]
            pltpu.sync_copy(obuf, out_hbm.at[pl.ds(start_row + i * 16, 16), :])
            return c_

        lax.fori_loop(0, n_hex, hex_body, 0)

    pl.when(wid < EXTRA)(lambda: run(BASE_HEX + 1))
    pl.when(wid >= EXTRA)(lambda: run(BASE_HEX))


def kernel(node_emb, segment_ids, weight, bias, scale):
    ids = segment_ids.astype(jnp.int32)
    mesh = plsc.VectorSubcoreMesh(core_axis_name="c", subcore_axis_name="s")

    stats = pl.kernel(
        _stats_body,
        mesh=mesh,
        out_type=(
            jax.ShapeDtypeStruct((NW, G, D), jnp.float32),
            jax.ShapeDtypeStruct((NW, G, D), jnp.float32),
            jax.ShapeDtypeStruct((NW, G, L), jnp.float32),
        ),
        scratch_types=[
            pltpu.VMEM((MAX_ROWS,), jnp.int32),
            pltpu.VMEM((16, D), jnp.float32),
            pltpu.VMEM((G, D), jnp.float32),
            pltpu.VMEM((G, D), jnp.float32),
            pltpu.VMEM((G, L), jnp.float32),
        ],
    )
    psum, psq, phist = stats(node_emb, ids)

    a_tab, c_tab = pl.pallas_call(
        _combine_body,
        out_shape=(
            jax.ShapeDtypeStruct((G, D), jnp.float32),
            jax.ShapeDtypeStruct((G, D), jnp.float32),
        ),
    )(psum, psq, phist,
      weight.reshape(1, D), bias.reshape(1, D), scale.reshape(1, D))

    norm = pl.kernel(
        _norm_body,
        mesh=mesh,
        out_type=jax.ShapeDtypeStruct((N, D), jnp.float32),
        scratch_types=[
            pltpu.VMEM((MAX_ROWS,), jnp.int32),
            pltpu.VMEM((16, D), jnp.float32),
            pltpu.VMEM((16, D), jnp.float32),
            pltpu.VMEM((G, D), jnp.float32),
            pltpu.VMEM((G, D), jnp.float32),
        ],
    )
    return norm(node_emb, ids, a_tab, c_tab)


# speedup vs baseline: 2.0686x; 1.1837x over previous
import jax
import jax.numpy as jnp
from jax import lax
from jax.experimental import pallas as pl
from jax.experimental.pallas import tpu as pltpu
from jax.experimental.pallas import tpu_sc as plsc

N = 100000
D = 128
G = 256
L = 16
NC = D // L
NW = 32

U = 32
TOT_U = N // U
BASE_U = TOT_U // NW
EXTRA = TOT_U - BASE_U * NW

MAX_ROWS = (BASE_U + 1) * U


def _wid():
    return lax.axis_index("s") * 2 + lax.axis_index("c")


def _start_row(wid):
    return (wid * BASE_U + jnp.minimum(wid, EXTRA)) * U


def _stats_process(ids_v, buf, acc, accsq, hist, i):
    ones = jnp.full((L,), 1.0, jnp.float32)
    for h in range(U // 16):
        idv = ids_v[pl.ds(i * U + h * 16, 16)]
        for r in range(16):
            s = idv[r]
            row = h * 16 + r
            for c in range(NC):
                sl = pl.ds(c * L, L)
                xv = buf[row, sl]
                plsc.addupdate(acc.at[s, sl], xv)
                plsc.addupdate(accsq.at[s, sl], xv * xv)
            plsc.addupdate(hist.at[s, :], ones)


def _stats_body(x_hbm, ids_hbm, psum_hbm, psq_hbm, phist_hbm,
                ids_v, xb0, xb1, acc, accsq, hist, sem_i, sem0, sem1):
    wid = _wid()
    start_row = _start_row(wid)

    zero16 = jnp.zeros((L,), jnp.float32)

    def run(n_u):
        n_rows = n_u * U
        pltpu.async_copy(ids_hbm.at[pl.ds(start_row, n_rows)],
                         ids_v.at[pl.ds(0, n_rows)], sem_i)

        def zbody(g, c_):
            for c in range(NC):
                acc[g, pl.ds(c * L, L)] = zero16
                accsq[g, pl.ds(c * L, L)] = zero16
            hist[g, :] = zero16
            return c_

        lax.fori_loop(0, G, zbody, 0)

        pltpu.make_async_copy(ids_hbm.at[pl.ds(start_row, n_rows)],
                              ids_v.at[pl.ds(0, n_rows)], sem_i).wait()

        def issue(u, buf, sem):
            pltpu.async_copy(x_hbm.at[pl.ds(start_row + u * U, U), :],
                             buf, sem)

        def wait(u, buf, sem):
            pltpu.make_async_copy(x_hbm.at[pl.ds(start_row + u * U, U), :],
                                  buf, sem).wait()

        issue(0, xb0, sem0)
        issue(1, xb1, sem1)
        n_pairs = n_u // 2

        def pbody(p, c_):
            i0 = 2 * p
            wait(i0, xb0, sem0)
            _stats_process(ids_v, xb0, acc, accsq, hist, i0)
            pl.when(i0 + 2 < n_u)(lambda: issue(i0 + 2, xb0, sem0))
            wait(i0 + 1, xb1, sem1)
            _stats_process(ids_v, xb1, acc, accsq, hist, i0 + 1)
            pl.when(i0 + 3 < n_u)(lambda: issue(i0 + 3, xb1, sem1))
            return c_

        lax.fori_loop(0, n_pairs, pbody, 0)
        if n_u % 2:
            wait(n_u - 1, xb0, sem0)
            _stats_process(ids_v, xb0, acc, accsq, hist, n_u - 1)

    pl.when(wid < EXTRA)(lambda: run(BASE_U + 1))
    pl.when(wid >= EXTRA)(lambda: run(BASE_U))

    pltpu.sync_copy(acc, psum_hbm.at[wid])
    pltpu.sync_copy(accsq, psq_hbm.at[wid])
    pltpu.sync_copy(hist, phist_hbm.at[wid])


def _combine_body(psum_ref, psq_ref, phist_ref, w_ref, b_ref, s_ref,
                  a_ref, c_ref):
    sums = jnp.sum(psum_ref[...], axis=0)
    sqs = jnp.sum(psq_ref[...], axis=0)
    cnt = jnp.sum(phist_ref[...], axis=0)[:, :1]
    cnt = jnp.maximum(cnt, 1.0)
    mean = sums / cnt
    t = mean * s_ref[...]
    var_sum = sqs - 2.0 * t * sums + cnt * t * t
    var = jnp.maximum(var_sum, 0.0) / cnt
    rstd = lax.rsqrt(var + 1e-8)
    a = w_ref[...] * rstd
    a_ref[...] = a
    c_ref[...] = b_ref[...] - a * t


def _norm_process(ids_v, buf, obuf, av, cv, i):
    for h in range(U // 16):
        idv = ids_v[pl.ds(i * U + h * 16, 16)]
        for r in range(16):
            s = idv[r]
            row = h * 16 + r
            for c in range(NC):
                sl = pl.ds(c * L, L)
                obuf[row, sl] = buf[row, sl] * av[s, sl] + cv[s, sl]


def _norm_body(x_hbm, ids_hbm, a_hbm, c_hbm, out_hbm,
               ids_v, xb0, xb1, ob0, ob1, av, cv,
               sem_i, sem0, sem1, semo0, semo1):
    wid = _wid()
    start_row = _start_row(wid)

    def run(n_u):
        n_rows = n_u * U
        pltpu.async_copy(ids_hbm.at[pl.ds(start_row, n_rows)],
                         ids_v.at[pl.ds(0, n_rows)], sem_i)
        pltpu.sync_copy(a_hbm, av)
        pltpu.sync_copy(c_hbm, cv)
        pltpu.make_async_copy(ids_hbm.at[pl.ds(start_row, n_rows)],
                              ids_v.at[pl.ds(0, n_rows)], sem_i).wait()

        def issue_in(u, buf, sem):
            pltpu.async_copy(x_hbm.at[pl.ds(start_row + u * U, U), :],
                             buf, sem)

        def wait_in(u, buf, sem):
            pltpu.make_async_copy(x_hbm.at[pl.ds(start_row + u * U, U), :],
                                  buf, sem).wait()

        def issue_out(u, buf, sem):
            pltpu.async_copy(buf, out_hbm.at[pl.ds(start_row + u * U, U), :],
                             sem)

        def wait_out(u, buf, sem):
            pltpu.make_async_copy(buf,
                                  out_hbm.at[pl.ds(start_row + u * U, U), :],
                                  sem).wait()

        issue_in(0, xb0, sem0)
        issue_in(1, xb1, sem1)
        n_pairs = n_u // 2

        def pbody(p, c_):
            i0 = 2 * p
            wait_in(i0, xb0, sem0)
            pl.when(p > 0)(lambda: wait_out(i0, ob0, semo0))
            _norm_process(ids_v, xb0, ob0, av, cv, i0)
            issue_out(i0, ob0, semo0)
            pl.when(i0 + 2 < n_u)(lambda: issue_in(i0 + 2, xb0, sem0))
            wait_in(i0 + 1, xb1, sem1)
            pl.when(p > 0)(lambda: wait_out(i0 + 1, ob1, semo1))
            _norm_process(ids_v, xb1, ob1, av, cv, i0 + 1)
            issue_out(i0 + 1, ob1, semo1)
            pl.when(i0 + 3 < n_u)(lambda: issue_in(i0 + 3, xb1, sem1))
            return c_

        lax.fori_loop(0, n_pairs, pbody, 0)
        if n_u % 2:
            wait_in(n_u - 1, xb0, sem0)
            wait_out(n_u - 1, ob0, semo0)
            _norm_process(ids_v, xb0, ob0, av, cv, n_u - 1)
            issue_out(n_u - 1, ob0, semo0)
        wait_out(0, ob0, semo0)
        wait_out(0, ob1, semo1)

    pl.when(wid < EXTRA)(lambda: run(BASE_U + 1))
    pl.when(wid >= EXTRA)(lambda: run(BASE_U))


def kernel(node_emb, segment_ids, weight, bias, scale):
    ids = segment_ids.astype(jnp.int32)
    mesh = plsc.VectorSubcoreMesh(core_axis_name="c", subcore_axis_name="s")

    stats = pl.kernel(
        _stats_body,
        mesh=mesh,
        out_type=(
            jax.ShapeDtypeStruct((NW, G, D), jnp.float32),
            jax.ShapeDtypeStruct((NW, G, D), jnp.float32),
            jax.ShapeDtypeStruct((NW, G, L), jnp.float32),
        ),
        scratch_types=[
            pltpu.VMEM((MAX_ROWS,), jnp.int32),
            pltpu.VMEM((U, D), jnp.float32),
            pltpu.VMEM((U, D), jnp.float32),
            pltpu.VMEM((G, D), jnp.float32),
            pltpu.VMEM((G, D), jnp.float32),
            pltpu.VMEM((G, L), jnp.float32),
            pltpu.SemaphoreType.DMA,
            pltpu.SemaphoreType.DMA,
            pltpu.SemaphoreType.DMA,
        ],
    )
    psum, psq, phist = stats(node_emb, ids)

    a_tab, c_tab = pl.pallas_call(
        _combine_body,
        out_shape=(
            jax.ShapeDtypeStruct((G, D), jnp.float32),
            jax.ShapeDtypeStruct((G, D), jnp.float32),
        ),
    )(psum, psq, phist,
      weight.reshape(1, D), bias.reshape(1, D), scale.reshape(1, D))

    norm = pl.kernel(
        _norm_body,
        mesh=mesh,
        out_type=jax.ShapeDtypeStruct((N, D), jnp.float32),
        scratch_types=[
            pltpu.VMEM((MAX_ROWS,), jnp.int32),
            pltpu.VMEM((U, D), jnp.float32),
            pltpu.VMEM((U, D), jnp.float32),
            pltpu.VMEM((U, D), jnp.float32),
            pltpu.VMEM((U, D), jnp.float32),
            pltpu.VMEM((G, D), jnp.float32),
            pltpu.VMEM((G, D), jnp.float32),
            pltpu.SemaphoreType.DMA,
            pltpu.SemaphoreType.DMA,
            pltpu.SemaphoreType.DMA,
            pltpu.SemaphoreType.DMA,
            pltpu.SemaphoreType.DMA,
        ],
    )
    return norm(node_emb, ids, a_tab, c_tab)


# speedup vs baseline: 6.0209x; 2.9107x over previous
import jax
import jax.numpy as jnp
from jax import lax
from jax.experimental import pallas as pl
from jax.experimental.pallas import tpu as pltpu
from jax.experimental.pallas import tpu_sc as plsc

N = 100000
D = 128
G = 256
L = 16
NC = D // L
NW = 32

U = 32
TOT_U = N // U
BASE_U = TOT_U // NW
EXTRA = TOT_U - BASE_U * NW

MAX_ROWS = (BASE_U + 1) * U


def _wid():
    return lax.axis_index("s") * 2 + lax.axis_index("c")


def _start_row(wid):
    return (wid * BASE_U + jnp.minimum(wid, EXTRA)) * U


def _tree_sum(vs):
    vs = list(vs)
    while len(vs) > 1:
        nxt = [vs[j] + vs[j + 1] for j in range(0, len(vs) - 1, 2)]
        if len(vs) % 2:
            nxt.append(vs[-1])
        vs = nxt
    return vs[0]


def _stats_process(ids_v, buf, acc, accsq, hist, i):
    ones = jnp.full((L,), 1.0, jnp.float32)
    sixteen = jnp.full((L,), 16.0, jnp.float32)

    def gbody(g, c_):
        goff = g * 16
        idv = ids_v[pl.ds(i * U + goff, 16)]
        s0 = idv[0]
        s15 = idv[15]

        def fast():
            for c in range(NC):
                sl = pl.ds(c * L, L)
                vals = [buf[goff + r, sl] for r in range(16)]
                plsc.addupdate(acc.at[s0, sl], _tree_sum(vals))
                plsc.addupdate(accsq.at[s0, sl],
                               _tree_sum([v * v for v in vals]))
            plsc.addupdate(hist.at[s0, :], sixteen)

        def slow():
            for r in range(16):
                s = idv[r]
                row = goff + r
                for c in range(NC):
                    sl = pl.ds(c * L, L)
                    xv = buf[row, sl]
                    plsc.addupdate(acc.at[s, sl], xv)
                    plsc.addupdate(accsq.at[s, sl], xv * xv)
                plsc.addupdate(hist.at[s, :], ones)

        pl.when(s0 == s15)(fast)
        pl.when(s0 != s15)(slow)
        return c_

    lax.fori_loop(0, U // 16, gbody, 0)


def _stats_body(x_hbm, ids_hbm, psum_hbm, psq_hbm, phist_hbm,
                ids_v, xb0, xb1, acc, accsq, hist, sem_i, sem0, sem1):
    wid = _wid()
    start_row = _start_row(wid)

    zero16 = jnp.zeros((L,), jnp.float32)

    def run(n_u):
        n_rows = n_u * U
        pltpu.async_copy(ids_hbm.at[pl.ds(start_row, n_rows)],
                         ids_v.at[pl.ds(0, n_rows)], sem_i)

        def zbody(g, c_):
            for c in range(NC):
                acc[g, pl.ds(c * L, L)] = zero16
                accsq[g, pl.ds(c * L, L)] = zero16
            hist[g, :] = zero16
            return c_

        lax.fori_loop(0, G, zbody, 0)

        pltpu.make_async_copy(ids_hbm.at[pl.ds(start_row, n_rows)],
                              ids_v.at[pl.ds(0, n_rows)], sem_i).wait()

        def issue(u, buf, sem):
            pltpu.async_copy(x_hbm.at[pl.ds(start_row + u * U, U), :],
                             buf, sem)

        def wait(u, buf, sem):
            pltpu.make_async_copy(x_hbm.at[pl.ds(start_row + u * U, U), :],
                                  buf, sem).wait()

        issue(0, xb0, sem0)
        issue(1, xb1, sem1)
        n_pairs = (n_u + 1) // 2

        def pbody(p, c_):
            i0 = 2 * p
            wait(i0, xb0, sem0)
            _stats_process(ids_v, xb0, acc, accsq, hist, i0)
            pl.when(i0 + 2 < n_u)(lambda: issue(i0 + 2, xb0, sem0))

            def second():
                wait(i0 + 1, xb1, sem1)
                _stats_process(ids_v, xb1, acc, accsq, hist, i0 + 1)
                pl.when(i0 + 3 < n_u)(lambda: issue(i0 + 3, xb1, sem1))

            pl.when(i0 + 1 < n_u)(second)
            return c_

        lax.fori_loop(0, n_pairs, pbody, 0)

    pl.when(wid < EXTRA)(lambda: run(BASE_U + 1))
    pl.when(wid >= EXTRA)(lambda: run(BASE_U))

    pltpu.sync_copy(acc, psum_hbm.at[wid])
    pltpu.sync_copy(accsq, psq_hbm.at[wid])
    pltpu.sync_copy(hist, phist_hbm.at[wid])


def _combine_body(psum_ref, psq_ref, phist_ref, w_ref, b_ref, s_ref,
                  a_ref, c_ref):
    sums = jnp.sum(psum_ref[...], axis=0)
    sqs = jnp.sum(psq_ref[...], axis=0)
    cnt = jnp.sum(phist_ref[...], axis=0)[:, :1]
    cnt = jnp.maximum(cnt, 1.0)
    mean = sums / cnt
    t = mean * s_ref[...]
    var_sum = sqs - 2.0 * t * sums + cnt * t * t
    var = jnp.maximum(var_sum, 0.0) / cnt
    rstd = lax.rsqrt(var + 1e-8)
    a = w_ref[...] * rstd
    a_ref[...] = a
    c_ref[...] = b_ref[...] - a * t


def _norm_process(ids_v, buf, obuf, av, cv, i):
    def gbody(g, c_):
        goff = g * 16
        idv = ids_v[pl.ds(i * U + goff, 16)]
        s0 = idv[0]
        s15 = idv[15]

        def fast():
            for c in range(NC):
                sl = pl.ds(c * L, L)
                a_c = av[s0, sl]
                c_c = cv[s0, sl]
                for r in range(16):
                    row = goff + r
                    obuf[row, sl] = buf[row, sl] * a_c + c_c

        def slow():
            for r in range(16):
                s = idv[r]
                row = goff + r
                for c in range(NC):
                    sl = pl.ds(c * L, L)
                    obuf[row, sl] = buf[row, sl] * av[s, sl] + cv[s, sl]

        pl.when(s0 == s15)(fast)
        pl.when(s0 != s15)(slow)
        return c_

    lax.fori_loop(0, U // 16, gbody, 0)


def _norm_body(x_hbm, ids_hbm, a_hbm, c_hbm, out_hbm,
               ids_v, xb0, xb1, ob0, ob1, av, cv,
               sem_i, sem0, sem1, semo0, semo1):
    wid = _wid()
    start_row = _start_row(wid)

    def run(n_u):
        n_rows = n_u * U
        pltpu.async_copy(ids_hbm.at[pl.ds(start_row, n_rows)],
                         ids_v.at[pl.ds(0, n_rows)], sem_i)
        pltpu.sync_copy(a_hbm, av)
        pltpu.sync_copy(c_hbm, cv)
        pltpu.make_async_copy(ids_hbm.at[pl.ds(start_row, n_rows)],
                              ids_v.at[pl.ds(0, n_rows)], sem_i).wait()

        def issue_in(u, buf, sem):
            pltpu.async_copy(x_hbm.at[pl.ds(start_row + u * U, U), :],
                             buf, sem)

        def wait_in(u, buf, sem):
            pltpu.make_async_copy(x_hbm.at[pl.ds(start_row + u * U, U), :],
                                  buf, sem).wait()

        def issue_out(u, buf, sem):
            pltpu.async_copy(buf, out_hbm.at[pl.ds(start_row + u * U, U), :],
                             sem)

        def wait_out(u, buf, sem):
            pltpu.make_async_copy(buf,
                                  out_hbm.at[pl.ds(start_row + u * U, U), :],
                                  sem).wait()

        issue_in(0, xb0, sem0)
        issue_in(1, xb1, sem1)
        n_pairs = (n_u + 1) // 2

        def pbody(p, c_):
            i0 = 2 * p
            wait_in(i0, xb0, sem0)
            pl.when(p > 0)(lambda: wait_out(i0, ob0, semo0))
            _norm_process(ids_v, xb0, ob0, av, cv, i0)
            issue_out(i0, ob0, semo0)
            pl.when(i0 + 2 < n_u)(lambda: issue_in(i0 + 2, xb0, sem0))

            def second():
                wait_in(i0 + 1, xb1, sem1)
                pl.when(p > 0)(lambda: wait_out(i0 + 1, ob1, semo1))
                _norm_process(ids_v, xb1, ob1, av, cv, i0 + 1)
                issue_out(i0 + 1, ob1, semo1)
                pl.when(i0 + 3 < n_u)(lambda: issue_in(i0 + 3, xb1, sem1))

            pl.when(i0 + 1 < n_u)(second)
            return c_

        lax.fori_loop(0, n_pairs, pbody, 0)
        wait_out(0, ob0, semo0)
        wait_out(0, ob1, semo1)

    pl.when(wid < EXTRA)(lambda: run(BASE_U + 1))
    pl.when(wid >= EXTRA)(lambda: run(BASE_U))


def kernel(node_emb, segment_ids, weight, bias, scale):
    ids = segment_ids.astype(jnp.int32)
    mesh = plsc.VectorSubcoreMesh(core_axis_name="c", subcore_axis_name="s")

    stats = pl.kernel(
        _stats_body,
        mesh=mesh,
        out_type=(
            jax.ShapeDtypeStruct((NW, G, D), jnp.float32),
            jax.ShapeDtypeStruct((NW, G, D), jnp.float32),
            jax.ShapeDtypeStruct((NW, G, L), jnp.float32),
        ),
        scratch_types=[
            pltpu.VMEM((MAX_ROWS,), jnp.int32),
            pltpu.VMEM((U, D), jnp.float32),
            pltpu.VMEM((U, D), jnp.float32),
            pltpu.VMEM((G, D), jnp.float32),
            pltpu.VMEM((G, D), jnp.float32),
            pltpu.VMEM((G, L), jnp.float32),
            pltpu.SemaphoreType.DMA,
            pltpu.SemaphoreType.DMA,
            pltpu.SemaphoreType.DMA,
        ],
    )
    psum, psq, phist = stats(node_emb, ids)

    a_tab, c_tab = pl.pallas_call(
        _combine_body,
        out_shape=(
            jax.ShapeDtypeStruct((G, D), jnp.float32),
            jax.ShapeDtypeStruct((G, D), jnp.float32),
        ),
    )(psum, psq, phist,
      weight.reshape(1, D), bias.reshape(1, D), scale.reshape(1, D))

    norm = pl.kernel(
        _norm_body,
        mesh=mesh,
        out_type=jax.ShapeDtypeStruct((N, D), jnp.float32),
        scratch_types=[
            pltpu.VMEM((MAX_ROWS,), jnp.int32),
            pltpu.VMEM((U, D), jnp.float32),
            pltpu.VMEM((U, D), jnp.float32),
            pltpu.VMEM((U, D), jnp.float32),
            pltpu.VMEM((U, D), jnp.float32),
            pltpu.VMEM((G, D), jnp.float32),
            pltpu.VMEM((G, D), jnp.float32),
            pltpu.SemaphoreType.DMA,
            pltpu.SemaphoreType.DMA,
            pltpu.SemaphoreType.DMA,
            pltpu.SemaphoreType.DMA,
            pltpu.SemaphoreType.DMA,
        ],
    )
    return norm(node_emb, ids, a_tab, c_tab)


# speedup vs baseline: 6.0660x; 1.0075x over previous
import jax
import jax.numpy as jnp
from jax import lax
from jax.experimental import pallas as pl
from jax.experimental.pallas import tpu as pltpu
from jax.experimental.pallas import tpu_sc as plsc

N = 100000
D = 128
G = 256
L = 16
NC = D // L
NW = 32

U = 32

B_TC = 2048
N_TC = 40960
N_SC = N - N_TC

TOT_U = N // U
BASE_U = TOT_U // NW
EXTRA = TOT_U - BASE_U * NW
MAX_ROWS = (BASE_U + 1) * U

TOT_US = N_SC // U
BASE_US = TOT_US // NW
EXTRA_S = TOT_US - BASE_US * NW
MAX_ROWS_S = (BASE_US + 1) * U


def _wid():
    return lax.axis_index("s") * 2 + lax.axis_index("c")


def _start_row(wid):
    return (wid * BASE_U + jnp.minimum(wid, EXTRA)) * U


def _start_row_s(wid):
    return N_TC + (wid * BASE_US + jnp.minimum(wid, EXTRA_S)) * U


def _tree_sum(vs):
    vs = list(vs)
    while len(vs) > 1:
        nxt = [vs[j] + vs[j + 1] for j in range(0, len(vs) - 1, 2)]
        if len(vs) % 2:
            nxt.append(vs[-1])
        vs = nxt
    return vs[0]


def _stats_process(ids_v, buf, acc, accsq, hist, i):
    ones = jnp.full((L,), 1.0, jnp.float32)
    sixteen = jnp.full((L,), 16.0, jnp.float32)

    def gbody(g, c_):
        goff = g * 16
        idv = ids_v[pl.ds(i * U + goff, 16)]
        s0 = idv[0]
        s15 = idv[15]

        def fast():
            for c in range(NC):
                sl = pl.ds(c * L, L)
                vals = [buf[goff + r, sl] for r in range(16)]
                plsc.addupdate(acc.at[s0, sl], _tree_sum(vals))
                plsc.addupdate(accsq.at[s0, sl],
                               _tree_sum([v * v for v in vals]))
            plsc.addupdate(hist.at[s0, :], sixteen)

        def slow():
            for r in range(16):
                s = idv[r]
                row = goff + r
                for c in range(NC):
                    sl = pl.ds(c * L, L)
                    xv = buf[row, sl]
                    plsc.addupdate(acc.at[s, sl], xv)
                    plsc.addupdate(accsq.at[s, sl], xv * xv)
                plsc.addupdate(hist.at[s, :], ones)

        pl.when(s0 == s15)(fast)
        pl.when(s0 != s15)(slow)
        return c_

    lax.fori_loop(0, U // 16, gbody, 0)


def _stats_body(x_hbm, ids_hbm, psum_hbm, psq_hbm, phist_hbm,
                ids_v, xb0, xb1, acc, accsq, hist, sem_i, sem0, sem1):
    wid = _wid()
    start_row = _start_row_s(wid)

    zero16 = jnp.zeros((L,), jnp.float32)

    def run(n_u):
        n_rows = n_u * U
        pltpu.async_copy(ids_hbm.at[pl.ds(start_row, n_rows)],
                         ids_v.at[pl.ds(0, n_rows)], sem_i)

        def zbody(g, c_):
            for c in range(NC):
                acc[g, pl.ds(c * L, L)] = zero16
                accsq[g, pl.ds(c * L, L)] = zero16
            hist[g, :] = zero16
            return c_

        lax.fori_loop(0, G, zbody, 0)

        pltpu.make_async_copy(ids_hbm.at[pl.ds(start_row, n_rows)],
                              ids_v.at[pl.ds(0, n_rows)], sem_i).wait()

        def issue(u, buf, sem):
            pltpu.async_copy(x_hbm.at[pl.ds(start_row + u * U, U), :],
                             buf, sem)

        def wait(u, buf, sem):
            pltpu.make_async_copy(x_hbm.at[pl.ds(start_row + u * U, U), :],
                                  buf, sem).wait()

        issue(0, xb0, sem0)
        issue(1, xb1, sem1)
        n_pairs = (n_u + 1) // 2

        def pbody(p, c_):
            i0 = 2 * p
            wait(i0, xb0, sem0)
            _stats_process(ids_v, xb0, acc, accsq, hist, i0)
            pl.when(i0 + 2 < n_u)(lambda: issue(i0 + 2, xb0, sem0))

            def second():
                wait(i0 + 1, xb1, sem1)
                _stats_process(ids_v, xb1, acc, accsq, hist, i0 + 1)
                pl.when(i0 + 3 < n_u)(lambda: issue(i0 + 3, xb1, sem1))

            pl.when(i0 + 1 < n_u)(second)
            return c_

        lax.fori_loop(0, n_pairs, pbody, 0)

    pl.when(wid < EXTRA_S)(lambda: run(BASE_US + 1))
    pl.when(wid >= EXTRA_S)(lambda: run(BASE_US))

    pltpu.sync_copy(acc, psum_hbm.at[wid])
    pltpu.sync_copy(accsq, psq_hbm.at[wid])
    pltpu.sync_copy(hist, phist_hbm.at[wid])


def _tc_stats_body(ids_ref, x_ref, psum_ref, psq_ref, cnt_ref):
    i = pl.program_id(0)
    x = x_ref[...]
    ids = ids_ref[...]
    seg = lax.broadcasted_iota(jnp.int32, (G, B_TC), 0)
    onehot = (ids == seg).astype(jnp.float32)
    dn = (((1,), (0,)), ((), ()))
    ps = lax.dot_general(onehot, x, dn,
                         precision=lax.Precision.HIGHEST,
                         preferred_element_type=jnp.float32)
    psq = lax.dot_general(onehot, x * x, dn,
                          precision=lax.Precision.HIGHEST,
                          preferred_element_type=jnp.float32)
    cnt = jnp.sum(onehot, axis=1, keepdims=True)

    @pl.when(i == 0)
    def _():
        psum_ref[...] = ps
        psq_ref[...] = psq
        cnt_ref[...] = cnt

    @pl.when(i > 0)
    def _():
        psum_ref[...] += ps
        psq_ref[...] += psq
        cnt_ref[...] += cnt


def _combine_body(psum_ref, psq_ref, phist_ref, tps_ref, tpsq_ref, tcnt_ref,
                  w_ref, b_ref, s_ref, a_ref, c_ref):
    sums = jnp.sum(psum_ref[...], axis=0) + tps_ref[...]
    sqs = jnp.sum(psq_ref[...], axis=0) + tpsq_ref[...]
    cnt = (jnp.sum(phist_ref[...], axis=0)[:, :1]
           + tcnt_ref[...])
    cnt = jnp.maximum(cnt, 1.0)
    mean = sums / cnt
    t = mean * s_ref[...]
    var_sum = sqs - 2.0 * t * sums + cnt * t * t
    var = jnp.maximum(var_sum, 0.0) / cnt
    rstd = lax.rsqrt(var + 1e-8)
    a = w_ref[...] * rstd
    a_ref[...] = a
    c_ref[...] = b_ref[...] - a * t


def _norm_process(ids_v, buf, obuf, av, cv, i):
    def gbody(g, c_):
        goff = g * 16
        idv = ids_v[pl.ds(i * U + goff, 16)]
        s0 = idv[0]
        s15 = idv[15]

        def fast():
            for c in range(NC):
                sl = pl.ds(c * L, L)
                a_c = av[s0, sl]
                c_c = cv[s0, sl]
                for r in range(16):
                    row = goff + r
                    obuf[row, sl] = buf[row, sl] * a_c + c_c

        def slow():
            for r in range(16):
                s = idv[r]
                row = goff + r
                for c in range(NC):
                    sl = pl.ds(c * L, L)
                    obuf[row, sl] = buf[row, sl] * av[s, sl] + cv[s, sl]

        pl.when(s0 == s15)(fast)
        pl.when(s0 != s15)(slow)
        return c_

    lax.fori_loop(0, U // 16, gbody, 0)


def _norm_body(x_hbm, ids_hbm, a_hbm, c_hbm, out_hbm,
               ids_v, xb0, xb1, ob0, ob1, av, cv,
               sem_i, sem0, sem1, semo0, semo1):
    wid = _wid()
    start_row = _start_row(wid)

    def run(n_u):
        n_rows = n_u * U
        pltpu.async_copy(ids_hbm.at[pl.ds(start_row, n_rows)],
                         ids_v.at[pl.ds(0, n_rows)], sem_i)
        pltpu.sync_copy(a_hbm, av)
        pltpu.sync_copy(c_hbm, cv)
        pltpu.make_async_copy(ids_hbm.at[pl.ds(start_row, n_rows)],
                              ids_v.at[pl.ds(0, n_rows)], sem_i).wait()

        def issue_in(u, buf, sem):
            pltpu.async_copy(x_hbm.at[pl.ds(start_row + u * U, U), :],
                             buf, sem)

        def wait_in(u, buf, sem):
            pltpu.make_async_copy(x_hbm.at[pl.ds(start_row + u * U, U), :],
                                  buf, sem).wait()

        def issue_out(u, buf, sem):
            pltpu.async_copy(buf, out_hbm.at[pl.ds(start_row + u * U, U), :],
                             sem)

        def wait_out(u, buf, sem):
            pltpu.make_async_copy(buf,
                                  out_hbm.at[pl.ds(start_row + u * U, U), :],
                                  sem).wait()

        issue_in(0, xb0, sem0)
        issue_in(1, xb1, sem1)
        n_pairs = (n_u + 1) // 2

        def pbody(p, c_):
            i0 = 2 * p
            wait_in(i0, xb0, sem0)
            pl.when(p > 0)(lambda: wait_out(i0, ob0, semo0))
            _norm_process(ids_v, xb0, ob0, av, cv, i0)
            issue_out(i0, ob0, semo0)
            pl.when(i0 + 2 < n_u)(lambda: issue_in(i0 + 2, xb0, sem0))

            def second():
                wait_in(i0 + 1, xb1, sem1)
                pl.when(p > 0)(lambda: wait_out(i0 + 1, ob1, semo1))
                _norm_process(ids_v, xb1, ob1, av, cv, i0 + 1)
                issue_out(i0 + 1, ob1, semo1)
                pl.when(i0 + 3 < n_u)(lambda: issue_in(i0 + 3, xb1, sem1))

            pl.when(i0 + 1 < n_u)(second)
            return c_

        lax.fori_loop(0, n_pairs, pbody, 0)
        wait_out(0, ob0, semo0)
        wait_out(0, ob1, semo1)

    pl.when(wid < EXTRA)(lambda: run(BASE_U + 1))
    pl.when(wid >= EXTRA)(lambda: run(BASE_U))


def kernel(node_emb, segment_ids, weight, bias, scale):
    ids = segment_ids.astype(jnp.int32)
    mesh = plsc.VectorSubcoreMesh(core_axis_name="c", subcore_axis_name="s")

    stats = pl.kernel(
        _stats_body,
        mesh=mesh,
        out_type=(
            jax.ShapeDtypeStruct((NW, G, D), jnp.float32),
            jax.ShapeDtypeStruct((NW, G, D), jnp.float32),
            jax.ShapeDtypeStruct((NW, G, L), jnp.float32),
        ),
        scratch_types=[
            pltpu.VMEM((MAX_ROWS_S,), jnp.int32),
            pltpu.VMEM((U, D), jnp.float32),
            pltpu.VMEM((U, D), jnp.float32),
            pltpu.VMEM((G, D), jnp.float32),
            pltpu.VMEM((G, D), jnp.float32),
            pltpu.VMEM((G, L), jnp.float32),
            pltpu.SemaphoreType.DMA,
            pltpu.SemaphoreType.DMA,
            pltpu.SemaphoreType.DMA,
        ],
    )
    psum, psq, phist = stats(node_emb, ids)

    tps, tpsq, tcnt = pl.pallas_call(
        _tc_stats_body,
        grid=(N_TC // B_TC,),
        in_specs=[
            pl.BlockSpec((1, B_TC), lambda i: (0, i)),
            pl.BlockSpec((B_TC, D), lambda i: (i, 0)),
        ],
        out_specs=(
            pl.BlockSpec((G, D), lambda i: (0, 0)),
            pl.BlockSpec((G, D), lambda i: (0, 0)),
            pl.BlockSpec((G, 1), lambda i: (0, 0)),
        ),
        out_shape=(
            jax.ShapeDtypeStruct((G, D), jnp.float32),
            jax.ShapeDtypeStruct((G, D), jnp.float32),
            jax.ShapeDtypeStruct((G, 1), jnp.float32),
        ),
    )(ids.reshape(1, N), node_emb)

    a_tab, c_tab = pl.pallas_call(
        _combine_body,
        out_shape=(
            jax.ShapeDtypeStruct((G, D), jnp.float32),
            jax.ShapeDtypeStruct((G, D), jnp.float32),
        ),
    )(psum, psq, phist, tps, tpsq, tcnt,
      weight.reshape(1, D), bias.reshape(1, D), scale.reshape(1, D))

    norm = pl.kernel(
        _norm_body,
        mesh=mesh,
        out_type=jax.ShapeDtypeStruct((N, D), jnp.float32),
        scratch_types=[
            pltpu.VMEM((MAX_ROWS,), jnp.int32),
            pltpu.VMEM((U, D), jnp.float32),
            pltpu.VMEM((U, D), jnp.float32),
            pltpu.VMEM((U, D), jnp.float32),
            pltpu.VMEM((U, D), jnp.float32),
            pltpu.VMEM((G, D), jnp.float32),
            pltpu.VMEM((G, D), jnp.float32),
            pltpu.SemaphoreType.DMA,
            pltpu.SemaphoreType.DMA,
            pltpu.SemaphoreType.DMA,
            pltpu.SemaphoreType.DMA,
            pltpu.SemaphoreType.DMA,
        ],
    )
    return norm(node_emb, ids, a_tab, c_tab)


# speedup vs baseline: 6.8529x; 1.1297x over previous
import jax
import jax.numpy as jnp
from jax import lax
from jax.experimental import pallas as pl
from jax.experimental.pallas import tpu as pltpu
from jax.experimental.pallas import tpu_sc as plsc

N = 100000
D = 128
G = 256
L = 16
NC = D // L
NW = 32

U = 32

B_TC = 2048
N_TC = 40960
N_SC = N - N_TC

TOT_U = N // U
BASE_U = TOT_U // NW
EXTRA = TOT_U - BASE_U * NW
MAX_ROWS = (BASE_U + 1) * U

TOT_US = N_SC // U
BASE_US = TOT_US // NW
EXTRA_S = TOT_US - BASE_US * NW
MAX_ROWS_S = (BASE_US + 1) * U


def _wid():
    return lax.axis_index("s") * 2 + lax.axis_index("c")


def _start_row(wid):
    return (wid * BASE_U + jnp.minimum(wid, EXTRA)) * U


def _start_row_s(wid):
    return N_TC + (wid * BASE_US + jnp.minimum(wid, EXTRA_S)) * U


def _tree_sum(vs):
    vs = list(vs)
    while len(vs) > 1:
        nxt = [vs[j] + vs[j + 1] for j in range(0, len(vs) - 1, 2)]
        if len(vs) % 2:
            nxt.append(vs[-1])
        vs = nxt
    return vs[0]


def _stats_process(ids_v, buf, acc, accsq, hist, i):
    ones = jnp.full((L,), 1.0, jnp.float32)
    sixteen = jnp.full((L,), 16.0, jnp.float32)

    def gbody(g, c_):
        goff = g * 16
        idv = ids_v[pl.ds(i * U + goff, 16)]
        s0 = idv[0]
        s15 = idv[15]

        def fast():
            for c in range(NC):
                sl = pl.ds(c * L, L)
                vals = [buf[goff + r, sl] for r in range(16)]
                plsc.addupdate(acc.at[s0, sl], _tree_sum(vals))
                plsc.addupdate(accsq.at[s0, sl],
                               _tree_sum([v * v for v in vals]))
            plsc.addupdate(hist.at[s0, :], sixteen)

        def slow():
            for r in range(16):
                s = idv[r]
                row = goff + r
                for c in range(NC):
                    sl = pl.ds(c * L, L)
                    xv = buf[row, sl]
                    plsc.addupdate(acc.at[s, sl], xv)
                    plsc.addupdate(accsq.at[s, sl], xv * xv)
                plsc.addupdate(hist.at[s, :], ones)

        pl.when(s0 == s15)(fast)
        pl.when(s0 != s15)(slow)
        return c_

    lax.fori_loop(0, U // 16, gbody, 0)


def _stats_body(x_hbm, ids_hbm, psum_hbm, psq_hbm, phist_hbm,
                ids_v, xb0, xb1, acc, accsq, hist, sem_i, sem0, sem1):
    wid = _wid()
    start_row = _start_row_s(wid)

    zero16 = jnp.zeros((L,), jnp.float32)

    def run(n_u):
        n_rows = n_u * U
        pltpu.async_copy(ids_hbm.at[pl.ds(start_row, n_rows)],
                         ids_v.at[pl.ds(0, n_rows)], sem_i)

        def zbody(g, c_):
            for c in range(NC):
                acc[g, pl.ds(c * L, L)] = zero16
                accsq[g, pl.ds(c * L, L)] = zero16
            hist[g, :] = zero16
            return c_

        lax.fori_loop(0, G, zbody, 0)

        pltpu.make_async_copy(ids_hbm.at[pl.ds(start_row, n_rows)],
                              ids_v.at[pl.ds(0, n_rows)], sem_i).wait()

        def issue(u, buf, sem):
            pltpu.async_copy(x_hbm.at[pl.ds(start_row + u * U, U), :],
                             buf, sem)

        def wait(u, buf, sem):
            pltpu.make_async_copy(x_hbm.at[pl.ds(start_row + u * U, U), :],
                                  buf, sem).wait()

        issue(0, xb0, sem0)
        issue(1, xb1, sem1)
        n_pairs = (n_u + 1) // 2

        def pbody(p, c_):
            i0 = 2 * p
            wait(i0, xb0, sem0)
            _stats_process(ids_v, xb0, acc, accsq, hist, i0)
            pl.when(i0 + 2 < n_u)(lambda: issue(i0 + 2, xb0, sem0))

            def second():
                wait(i0 + 1, xb1, sem1)
                _stats_process(ids_v, xb1, acc, accsq, hist, i0 + 1)
                pl.when(i0 + 3 < n_u)(lambda: issue(i0 + 3, xb1, sem1))

            pl.when(i0 + 1 < n_u)(second)
            return c_

        lax.fori_loop(0, n_pairs, pbody, 0)

    pl.when(wid < EXTRA_S)(lambda: run(BASE_US + 1))
    pl.when(wid >= EXTRA_S)(lambda: run(BASE_US))

    pltpu.sync_copy(acc, psum_hbm.at[wid])
    pltpu.sync_copy(accsq, psq_hbm.at[wid])
    pltpu.sync_copy(hist, phist_hbm.at[wid])


def _tc_stats_body(ids_ref, x_ref, psum_ref, psq_ref, cnt_ref):
    i = pl.program_id(0)
    x = x_ref[...]
    ids = ids_ref[...]
    seg = lax.broadcasted_iota(jnp.int32, (G, B_TC), 0)
    onehot = (ids == seg).astype(jnp.float32)
    oh_bf = onehot.astype(jnp.bfloat16)
    dn = (((1,), (0,)), ((), ()))

    def segdot(v):
        vh = v.astype(jnp.bfloat16)
        vl = (v - vh.astype(jnp.float32)).astype(jnp.bfloat16)
        ph = lax.dot_general(oh_bf, vh, dn,
                             preferred_element_type=jnp.float32)
        return ph + lax.dot_general(oh_bf, vl, dn,
                                    preferred_element_type=jnp.float32)

    ps = segdot(x)
    psq = segdot(x * x)
    cnt = jnp.sum(onehot, axis=1, keepdims=True)

    @pl.when(i == 0)
    def _():
        psum_ref[...] = ps
        psq_ref[...] = psq
        cnt_ref[...] = cnt

    @pl.when(i > 0)
    def _():
        psum_ref[...] += ps
        psq_ref[...] += psq
        cnt_ref[...] += cnt


def _combine_body(psum_ref, psq_ref, phist_ref, tps_ref, tpsq_ref, tcnt_ref,
                  w_ref, b_ref, s_ref, a_ref, c_ref):
    sums = jnp.sum(psum_ref[...], axis=0) + tps_ref[...]
    sqs = jnp.sum(psq_ref[...], axis=0) + tpsq_ref[...]
    cnt = (jnp.sum(phist_ref[...], axis=0)[:, :1]
           + tcnt_ref[...])
    cnt = jnp.maximum(cnt, 1.0)
    mean = sums / cnt
    t = mean * s_ref[...]
    var_sum = sqs - 2.0 * t * sums + cnt * t * t
    var = jnp.maximum(var_sum, 0.0) / cnt
    rstd = lax.rsqrt(var + 1e-8)
    a = w_ref[...] * rstd
    a_ref[...] = a
    c_ref[...] = b_ref[...] - a * t


def _norm_process(ids_v, buf, obuf, av, cv, i):
    def gbody(g, c_):
        goff = g * 16
        idv = ids_v[pl.ds(i * U + goff, 16)]
        s0 = idv[0]
        s15 = idv[15]

        def fast():
            for c in range(NC):
                sl = pl.ds(c * L, L)
                a_c = av[s0, sl]
                c_c = cv[s0, sl]
                for r in range(16):
                    row = goff + r
                    obuf[row, sl] = buf[row, sl] * a_c + c_c

        def slow():
            for r in range(16):
                s = idv[r]
                row = goff + r
                for c in range(NC):
                    sl = pl.ds(c * L, L)
                    obuf[row, sl] = buf[row, sl] * av[s, sl] + cv[s, sl]

        pl.when(s0 == s15)(fast)
        pl.when(s0 != s15)(slow)
        return c_

    lax.fori_loop(0, U // 16, gbody, 0)


def _norm_body(x_hbm, ids_hbm, a_hbm, c_hbm, out_hbm,
               ids_v, xb0, xb1, ob0, ob1, av, cv,
               sem_i, sem0, sem1, semo0, semo1):
    wid = _wid()
    start_row = _start_row(wid)

    def run(n_u):
        n_rows = n_u * U
        pltpu.async_copy(ids_hbm.at[pl.ds(start_row, n_rows)],
                         ids_v.at[pl.ds(0, n_rows)], sem_i)
        pltpu.sync_copy(a_hbm, av)
        pltpu.sync_copy(c_hbm, cv)
        pltpu.make_async_copy(ids_hbm.at[pl.ds(start_row, n_rows)],
                              ids_v.at[pl.ds(0, n_rows)], sem_i).wait()

        def issue_in(u, buf, sem):
            pltpu.async_copy(x_hbm.at[pl.ds(start_row + u * U, U), :],
                             buf, sem)

        def wait_in(u, buf, sem):
            pltpu.make_async_copy(x_hbm.at[pl.ds(start_row + u * U, U), :],
                                  buf, sem).wait()

        def issue_out(u, buf, sem):
            pltpu.async_copy(buf, out_hbm.at[pl.ds(start_row + u * U, U), :],
                             sem)

        def wait_out(u, buf, sem):
            pltpu.make_async_copy(buf,
                                  out_hbm.at[pl.ds(start_row + u * U, U), :],
                                  sem).wait()

        issue_in(0, xb0, sem0)
        issue_in(1, xb1, sem1)
        n_pairs = (n_u + 1) // 2

        def pbody(p, c_):
            i0 = 2 * p
            wait_in(i0, xb0, sem0)
            pl.when(p > 0)(lambda: wait_out(i0, ob0, semo0))
            _norm_process(ids_v, xb0, ob0, av, cv, i0)
            issue_out(i0, ob0, semo0)
            pl.when(i0 + 2 < n_u)(lambda: issue_in(i0 + 2, xb0, sem0))

            def second():
                wait_in(i0 + 1, xb1, sem1)
                pl.when(p > 0)(lambda: wait_out(i0 + 1, ob1, semo1))
                _norm_process(ids_v, xb1, ob1, av, cv, i0 + 1)
                issue_out(i0 + 1, ob1, semo1)
                pl.when(i0 + 3 < n_u)(lambda: issue_in(i0 + 3, xb1, sem1))

            pl.when(i0 + 1 < n_u)(second)
            return c_

        lax.fori_loop(0, n_pairs, pbody, 0)
        wait_out(0, ob0, semo0)
        wait_out(0, ob1, semo1)

    pl.when(wid < EXTRA)(lambda: run(BASE_U + 1))
    pl.when(wid >= EXTRA)(lambda: run(BASE_U))


def kernel(node_emb, segment_ids, weight, bias, scale):
    ids = segment_ids.astype(jnp.int32)
    mesh = plsc.VectorSubcoreMesh(core_axis_name="c", subcore_axis_name="s")

    stats = pl.kernel(
        _stats_body,
        mesh=mesh,
        out_type=(
            jax.ShapeDtypeStruct((NW, G, D), jnp.float32),
            jax.ShapeDtypeStruct((NW, G, D), jnp.float32),
            jax.ShapeDtypeStruct((NW, G, L), jnp.float32),
        ),
        scratch_types=[
            pltpu.VMEM((MAX_ROWS_S,), jnp.int32),
            pltpu.VMEM((U, D), jnp.float32),
            pltpu.VMEM((U, D), jnp.float32),
            pltpu.VMEM((G, D), jnp.float32),
            pltpu.VMEM((G, D), jnp.float32),
            pltpu.VMEM((G, L), jnp.float32),
            pltpu.SemaphoreType.DMA,
            pltpu.SemaphoreType.DMA,
            pltpu.SemaphoreType.DMA,
        ],
    )
    psum, psq, phist = stats(node_emb, ids)

    tps, tpsq, tcnt = pl.pallas_call(
        _tc_stats_body,
        grid=(N_TC // B_TC,),
        in_specs=[
            pl.BlockSpec((1, B_TC), lambda i: (0, i)),
            pl.BlockSpec((B_TC, D), lambda i: (i, 0)),
        ],
        out_specs=(
            pl.BlockSpec((G, D), lambda i: (0, 0)),
            pl.BlockSpec((G, D), lambda i: (0, 0)),
            pl.BlockSpec((G, 1), lambda i: (0, 0)),
        ),
        out_shape=(
            jax.ShapeDtypeStruct((G, D), jnp.float32),
            jax.ShapeDtypeStruct((G, D), jnp.float32),
            jax.ShapeDtypeStruct((G, 1), jnp.float32),
        ),
    )(ids.reshape(1, N), node_emb)

    a_tab, c_tab = pl.pallas_call(
        _combine_body,
        out_shape=(
            jax.ShapeDtypeStruct((G, D), jnp.float32),
            jax.ShapeDtypeStruct((G, D), jnp.float32),
        ),
    )(psum, psq, phist, tps, tpsq, tcnt,
      weight.reshape(1, D), bias.reshape(1, D), scale.reshape(1, D))

    norm = pl.kernel(
        _norm_body,
        mesh=mesh,
        out_type=jax.ShapeDtypeStruct((N, D), jnp.float32),
        scratch_types=[
            pltpu.VMEM((MAX_ROWS,), jnp.int32),
            pltpu.VMEM((U, D), jnp.float32),
            pltpu.VMEM((U, D), jnp.float32),
            pltpu.VMEM((U, D), jnp.float32),
            pltpu.VMEM((U, D), jnp.float32),
            pltpu.VMEM((G, D), jnp.float32),
            pltpu.VMEM((G, D), jnp.float32),
            pltpu.SemaphoreType.DMA,
            pltpu.SemaphoreType.DMA,
            pltpu.SemaphoreType.DMA,
            pltpu.SemaphoreType.DMA,
            pltpu.SemaphoreType.DMA,
        ],
    )
    return norm(node_emb, ids, a_tab, c_tab)


# speedup vs baseline: 7.1135x; 1.0380x over previous
import jax
import jax.numpy as jnp
from jax import lax
from jax.experimental import pallas as pl
from jax.experimental.pallas import tpu as pltpu
from jax.experimental.pallas import tpu_sc as plsc

N = 100000
D = 128
G = 256
L = 16
NC = D // L
NW = 32

U = 32

B_TC = 2048
N_TC = 53248
N_SC = N - N_TC

TOT_U = N // U
BASE_U = TOT_U // NW
EXTRA = TOT_U - BASE_U * NW
MAX_ROWS = (BASE_U + 1) * U

TOT_US = N_SC // U
BASE_US = TOT_US // NW
EXTRA_S = TOT_US - BASE_US * NW
MAX_ROWS_S = (BASE_US + 1) * U


def _wid():
    return lax.axis_index("s") * 2 + lax.axis_index("c")


def _start_row(wid):
    return (wid * BASE_U + jnp.minimum(wid, EXTRA)) * U


def _start_row_s(wid):
    return N_TC + (wid * BASE_US + jnp.minimum(wid, EXTRA_S)) * U


def _tree_sum(vs):
    vs = list(vs)
    while len(vs) > 1:
        nxt = [vs[j] + vs[j + 1] for j in range(0, len(vs) - 1, 2)]
        if len(vs) % 2:
            nxt.append(vs[-1])
        vs = nxt
    return vs[0]


def _stats_process(ids_v, buf, acc, accsq, hist, i):
    ones = jnp.full((L,), 1.0, jnp.float32)
    sixteen = jnp.full((L,), 16.0, jnp.float32)

    def gbody(g, c_):
        goff = g * 16
        idv = ids_v[pl.ds(i * U + goff, 16)]
        s0 = idv[0]
        s15 = idv[15]

        def fast():
            for c in range(NC):
                sl = pl.ds(c * L, L)
                vals = [buf[goff + r, sl] for r in range(16)]
                plsc.addupdate(acc.at[s0, sl], _tree_sum(vals))
                plsc.addupdate(accsq.at[s0, sl],
                               _tree_sum([v * v for v in vals]))
            plsc.addupdate(hist.at[s0, :], sixteen)

        def slow():
            for r in range(16):
                s = idv[r]
                row = goff + r
                for c in range(NC):
                    sl = pl.ds(c * L, L)
                    xv = buf[row, sl]
                    plsc.addupdate(acc.at[s, sl], xv)
                    plsc.addupdate(accsq.at[s, sl], xv * xv)
                plsc.addupdate(hist.at[s, :], ones)

        pl.when(s0 == s15)(fast)
        pl.when(s0 != s15)(slow)
        return c_

    lax.fori_loop(0, U // 16, gbody, 0)


def _stats_body(x_hbm, ids_hbm, psum_hbm, psq_hbm, phist_hbm,
                ids_v, xb0, xb1, acc, accsq, hist, sem_i, sem0, sem1):
    wid = _wid()
    start_row = _start_row_s(wid)

    zero16 = jnp.zeros((L,), jnp.float32)

    def run(n_u):
        n_rows = n_u * U
        pltpu.async_copy(ids_hbm.at[pl.ds(start_row, n_rows)],
                         ids_v.at[pl.ds(0, n_rows)], sem_i)

        def zbody(g, c_):
            for c in range(NC):
                acc[g, pl.ds(c * L, L)] = zero16
                accsq[g, pl.ds(c * L, L)] = zero16
            hist[g, :] = zero16
            return c_

        lax.fori_loop(0, G, zbody, 0)

        pltpu.make_async_copy(ids_hbm.at[pl.ds(start_row, n_rows)],
                              ids_v.at[pl.ds(0, n_rows)], sem_i).wait()

        def issue(u, buf, sem):
            pltpu.async_copy(x_hbm.at[pl.ds(start_row + u * U, U), :],
                             buf, sem)

        def wait(u, buf, sem):
            pltpu.make_async_copy(x_hbm.at[pl.ds(start_row + u * U, U), :],
                                  buf, sem).wait()

        issue(0, xb0, sem0)
        issue(1, xb1, sem1)
        n_pairs = (n_u + 1) // 2

        def pbody(p, c_):
            i0 = 2 * p
            wait(i0, xb0, sem0)
            _stats_process(ids_v, xb0, acc, accsq, hist, i0)
            pl.when(i0 + 2 < n_u)(lambda: issue(i0 + 2, xb0, sem0))

            def second():
                wait(i0 + 1, xb1, sem1)
                _stats_process(ids_v, xb1, acc, accsq, hist, i0 + 1)
                pl.when(i0 + 3 < n_u)(lambda: issue(i0 + 3, xb1, sem1))

            pl.when(i0 + 1 < n_u)(second)
            return c_

        lax.fori_loop(0, n_pairs, pbody, 0)

    pl.when(wid < EXTRA_S)(lambda: run(BASE_US + 1))
    pl.when(wid >= EXTRA_S)(lambda: run(BASE_US))

    pltpu.sync_copy(acc, psum_hbm.at[wid])
    pltpu.sync_copy(accsq, psq_hbm.at[wid])
    pltpu.sync_copy(hist, phist_hbm.at[wid])


def _tc_stats_body(ids_ref, x_ref, psum_ref, psq_ref, cnt_ref):
    i = pl.program_id(0)
    x = x_ref[...]
    ids = ids_ref[...]
    seg = lax.broadcasted_iota(jnp.int32, (G, B_TC), 0)
    onehot = (ids == seg).astype(jnp.float32)
    oh_bf = onehot.astype(jnp.bfloat16)
    dn = (((1,), (0,)), ((), ()))

    def segdot(v):
        vh = v.astype(jnp.bfloat16)
        vl = (v - vh.astype(jnp.float32)).astype(jnp.bfloat16)
        ph = lax.dot_general(oh_bf, vh, dn,
                             preferred_element_type=jnp.float32)
        return ph + lax.dot_general(oh_bf, vl, dn,
                                    preferred_element_type=jnp.float32)

    ps = segdot(x)
    psq = segdot(x * x)
    cnt = jnp.sum(onehot, axis=1, keepdims=True)

    @pl.when(i == 0)
    def _():
        psum_ref[...] = ps
        psq_ref[...] = psq
        cnt_ref[...] = cnt

    @pl.when(i > 0)
    def _():
        psum_ref[...] += ps
        psq_ref[...] += psq
        cnt_ref[...] += cnt


def _combine_body(psum_ref, psq_ref, phist_ref, tps_ref, tpsq_ref, tcnt_ref,
                  w_ref, b_ref, s_ref, a_ref, c_ref):
    sums = jnp.sum(psum_ref[...], axis=0) + tps_ref[...]
    sqs = jnp.sum(psq_ref[...], axis=0) + tpsq_ref[...]
    cnt = (jnp.sum(phist_ref[...], axis=0)[:, :1]
           + tcnt_ref[...])
    cnt = jnp.maximum(cnt, 1.0)
    mean = sums / cnt
    t = mean * s_ref[...]
    var_sum = sqs - 2.0 * t * sums + cnt * t * t
    var = jnp.maximum(var_sum, 0.0) / cnt
    rstd = lax.rsqrt(var + 1e-8)
    a = w_ref[...] * rstd
    a_ref[...] = a
    c_ref[...] = b_ref[...] - a * t


def _norm_process(ids_v, buf, obuf, av, cv, i):
    def gbody(g, c_):
        goff = g * 16
        idv = ids_v[pl.ds(i * U + goff, 16)]
        s0 = idv[0]
        s15 = idv[15]

        def fast():
            for c in range(NC):
                sl = pl.ds(c * L, L)
                a_c = av[s0, sl]
                c_c = cv[s0, sl]
                for r in range(16):
                    row = goff + r
                    obuf[row, sl] = buf[row, sl] * a_c + c_c

        def slow():
            for r in range(16):
                s = idv[r]
                row = goff + r
                for c in range(NC):
                    sl = pl.ds(c * L, L)
                    obuf[row, sl] = buf[row, sl] * av[s, sl] + cv[s, sl]

        pl.when(s0 == s15)(fast)
        pl.when(s0 != s15)(slow)
        return c_

    lax.fori_loop(0, U // 16, gbody, 0)


def _norm_body(x_hbm, ids_hbm, a_hbm, c_hbm, out_hbm,
               ids_v, xb0, xb1, ob0, ob1, av, cv,
               sem_i, sem0, sem1, semo0, semo1):
    wid = _wid()
    start_row = _start_row(wid)

    def run(n_u):
        n_rows = n_u * U
        pltpu.async_copy(ids_hbm.at[pl.ds(start_row, n_rows)],
                         ids_v.at[pl.ds(0, n_rows)], sem_i)
        pltpu.sync_copy(a_hbm, av)
        pltpu.sync_copy(c_hbm, cv)
        pltpu.make_async_copy(ids_hbm.at[pl.ds(start_row, n_rows)],
                              ids_v.at[pl.ds(0, n_rows)], sem_i).wait()

        def issue_in(u, buf, sem):
            pltpu.async_copy(x_hbm.at[pl.ds(start_row + u * U, U), :],
                             buf, sem)

        def wait_in(u, buf, sem):
            pltpu.make_async_copy(x_hbm.at[pl.ds(start_row + u * U, U), :],
                                  buf, sem).wait()

        def issue_out(u, buf, sem):
            pltpu.async_copy(buf, out_hbm.at[pl.ds(start_row + u * U, U), :],
                             sem)

        def wait_out(u, buf, sem):
            pltpu.make_async_copy(buf,
                                  out_hbm.at[pl.ds(start_row + u * U, U), :],
                                  sem).wait()

        issue_in(0, xb0, sem0)
        issue_in(1, xb1, sem1)
        n_pairs = (n_u + 1) // 2

        def pbody(p, c_):
            i0 = 2 * p
            wait_in(i0, xb0, sem0)
            pl.when(p > 0)(lambda: wait_out(i0, ob0, semo0))
            _norm_process(ids_v, xb0, ob0, av, cv, i0)
            issue_out(i0, ob0, semo0)
            pl.when(i0 + 2 < n_u)(lambda: issue_in(i0 + 2, xb0, sem0))

            def second():
                wait_in(i0 + 1, xb1, sem1)
                pl.when(p > 0)(lambda: wait_out(i0 + 1, ob1, semo1))
                _norm_process(ids_v, xb1, ob1, av, cv, i0 + 1)
                issue_out(i0 + 1, ob1, semo1)
                pl.when(i0 + 3 < n_u)(lambda: issue_in(i0 + 3, xb1, sem1))

            pl.when(i0 + 1 < n_u)(second)
            return c_

        lax.fori_loop(0, n_pairs, pbody, 0)
        wait_out(0, ob0, semo0)
        wait_out(0, ob1, semo1)

    pl.when(wid < EXTRA)(lambda: run(BASE_U + 1))
    pl.when(wid >= EXTRA)(lambda: run(BASE_U))


def kernel(node_emb, segment_ids, weight, bias, scale):
    ids = segment_ids.astype(jnp.int32)
    mesh = plsc.VectorSubcoreMesh(core_axis_name="c", subcore_axis_name="s")

    stats = pl.kernel(
        _stats_body,
        mesh=mesh,
        out_type=(
            jax.ShapeDtypeStruct((NW, G, D), jnp.float32),
            jax.ShapeDtypeStruct((NW, G, D), jnp.float32),
            jax.ShapeDtypeStruct((NW, G, L), jnp.float32),
        ),
        scratch_types=[
            pltpu.VMEM((MAX_ROWS_S,), jnp.int32),
            pltpu.VMEM((U, D), jnp.float32),
            pltpu.VMEM((U, D), jnp.float32),
            pltpu.VMEM((G, D), jnp.float32),
            pltpu.VMEM((G, D), jnp.float32),
            pltpu.VMEM((G, L), jnp.float32),
            pltpu.SemaphoreType.DMA,
            pltpu.SemaphoreType.DMA,
            pltpu.SemaphoreType.DMA,
        ],
    )
    psum, psq, phist = stats(node_emb, ids)

    tps, tpsq, tcnt = pl.pallas_call(
        _tc_stats_body,
        grid=(N_TC // B_TC,),
        in_specs=[
            pl.BlockSpec((1, B_TC), lambda i: (0, i)),
            pl.BlockSpec((B_TC, D), lambda i: (i, 0)),
        ],
        out_specs=(
            pl.BlockSpec((G, D), lambda i: (0, 0)),
            pl.BlockSpec((G, D), lambda i: (0, 0)),
            pl.BlockSpec((G, 1), lambda i: (0, 0)),
        ),
        out_shape=(
            jax.ShapeDtypeStruct((G, D), jnp.float32),
            jax.ShapeDtypeStruct((G, D), jnp.float32),
            jax.ShapeDtypeStruct((G, 1), jnp.float32),
        ),
    )(ids.reshape(1, N), node_emb)

    a_tab, c_tab = pl.pallas_call(
        _combine_body,
        out_shape=(
            jax.ShapeDtypeStruct((G, D), jnp.float32),
            jax.ShapeDtypeStruct((G, D), jnp.float32),
        ),
    )(psum, psq, phist, tps, tpsq, tcnt,
      weight.reshape(1, D), bias.reshape(1, D), scale.reshape(1, D))

    norm = pl.kernel(
        _norm_body,
        mesh=mesh,
        out_type=jax.ShapeDtypeStruct((N, D), jnp.float32),
        scratch_types=[
            pltpu.VMEM((MAX_ROWS,), jnp.int32),
            pltpu.VMEM((U, D), jnp.float32),
            pltpu.VMEM((U, D), jnp.float32),
            pltpu.VMEM((U, D), jnp.float32),
            pltpu.VMEM((U, D), jnp.float32),
            pltpu.VMEM((G, D), jnp.float32),
            pltpu.VMEM((G, D), jnp.float32),
            pltpu.SemaphoreType.DMA,
            pltpu.SemaphoreType.DMA,
            pltpu.SemaphoreType.DMA,
            pltpu.SemaphoreType.DMA,
            pltpu.SemaphoreType.DMA,
        ],
    )
    return norm(node_emb, ids, a_tab, c_tab)


# speedup vs baseline: 7.3790x; 1.0373x over previous
import jax
import jax.numpy as jnp
from jax import lax
from jax.experimental import pallas as pl
from jax.experimental.pallas import tpu as pltpu
from jax.experimental.pallas import tpu_sc as plsc

N = 100000
D = 128
G = 256
L = 16
NC = D // L
NW = 32

U = 32
WSEG = 64

B_TC = 2048
N_TC = 53248
N_SC = N - N_TC

UN = 32
TOT_U = N // UN
BASE_U = TOT_U // NW
EXTRA = TOT_U - BASE_U * NW
MAX_ROWS = (BASE_U + 1) * UN

TOT_US = N_SC // U
BASE_US = TOT_US // NW
EXTRA_S = TOT_US - BASE_US * NW
MAX_ROWS_S = (BASE_US + 1) * U


def _wid():
    return lax.axis_index("s") * 2 + lax.axis_index("c")


def _start_row(wid):
    return (wid * BASE_U + jnp.minimum(wid, EXTRA)) * UN


def _start_row_s(wid):
    return N_TC + (wid * BASE_US + jnp.minimum(wid, EXTRA_S)) * U


def _tree_sum(vs):
    vs = list(vs)
    while len(vs) > 1:
        nxt = [vs[j] + vs[j + 1] for j in range(0, len(vs) - 1, 2)]
        if len(vs) % 2:
            nxt.append(vs[-1])
        vs = nxt
    return vs[0]


def _stats_process(ids_v, buf, acc, accsq, hist, i):
    ones = jnp.full((L,), 1.0, jnp.float32)
    sixteen = jnp.full((L,), 16.0, jnp.float32)

    def gbody(g, c_):
        goff = g * 16
        idv = ids_v[pl.ds(i * U + goff, 16)]
        s0 = idv[0]
        s15 = idv[15]

        def fast():
            for c in range(NC):
                sl = pl.ds(c * L, L)
                vals = [buf[goff + r, sl] for r in range(16)]
                plsc.addupdate(acc.at[s0, sl], _tree_sum(vals))
                plsc.addupdate(accsq.at[s0, sl],
                               _tree_sum([v * v for v in vals]))
            plsc.addupdate(hist.at[s0, :], sixteen)

        def slow():
            for r in range(16):
                s = idv[r]
                row = goff + r
                for c in range(NC):
                    sl = pl.ds(c * L, L)
                    xv = buf[row, sl]
                    plsc.addupdate(acc.at[s, sl], xv)
                    plsc.addupdate(accsq.at[s, sl], xv * xv)
                plsc.addupdate(hist.at[s, :], ones)

        pl.when(s0 == s15)(fast)
        pl.when(s0 != s15)(slow)
        return c_

    lax.fori_loop(0, U // 16, gbody, 0)


def _stats_body(x_hbm, ids_hbm, psum_hbm, psq_hbm, phist_hbm,
                ids_v, xb0, xb1, acc, accsq, hist, sem_i, sem0, sem1):
    wid = _wid()
    start_row = _start_row_s(wid)

    zero16 = jnp.zeros((L,), jnp.float32)

    def run(n_u):
        n_rows = n_u * U
        pltpu.async_copy(ids_hbm.at[pl.ds(start_row, n_rows)],
                         ids_v.at[pl.ds(0, n_rows)], sem_i)

        def zbody(g, c_):
            for c in range(NC):
                acc[g, pl.ds(c * L, L)] = zero16
                accsq[g, pl.ds(c * L, L)] = zero16
            hist[g, :] = zero16
            return c_

        lax.fori_loop(0, G, zbody, 0)

        pltpu.make_async_copy(ids_hbm.at[pl.ds(start_row, n_rows)],
                              ids_v.at[pl.ds(0, n_rows)], sem_i).wait()

        def issue(u, buf, sem):
            pltpu.async_copy(x_hbm.at[pl.ds(start_row + u * U, U), :],
                             buf, sem)

        def wait(u, buf, sem):
            pltpu.make_async_copy(x_hbm.at[pl.ds(start_row + u * U, U), :],
                                  buf, sem).wait()

        issue(0, xb0, sem0)
        issue(1, xb1, sem1)
        n_pairs = (n_u + 1) // 2

        def pbody(p, c_):
            i0 = 2 * p
            wait(i0, xb0, sem0)
            _stats_process(ids_v, xb0, acc, accsq, hist, i0)
            pl.when(i0 + 2 < n_u)(lambda: issue(i0 + 2, xb0, sem0))

            def second():
                wait(i0 + 1, xb1, sem1)
                _stats_process(ids_v, xb1, acc, accsq, hist, i0 + 1)
                pl.when(i0 + 3 < n_u)(lambda: issue(i0 + 3, xb1, sem1))

            pl.when(i0 + 1 < n_u)(second)
            return c_

        lax.fori_loop(0, n_pairs, pbody, 0)

    pl.when(wid < EXTRA_S)(lambda: run(BASE_US + 1))
    pl.when(wid >= EXTRA_S)(lambda: run(BASE_US))

    pltpu.sync_copy(acc, psum_hbm.at[wid])
    pltpu.sync_copy(accsq, psq_hbm.at[wid])
    pltpu.sync_copy(hist, phist_hbm.at[wid])


def _tc_stats_body(ids_ref, x_ref, psum_ref, psq_ref, cnt_ref):
    i = pl.program_id(0)
    x = x_ref[...]
    ids = ids_ref[...]
    seg = lax.broadcasted_iota(jnp.int32, (G, B_TC), 0)
    onehot = (ids == seg).astype(jnp.float32)
    oh_bf = onehot.astype(jnp.bfloat16)
    dn = (((1,), (0,)), ((), ()))

    def segdot(v):
        vh = v.astype(jnp.bfloat16)
        vl = (v - vh.astype(jnp.float32)).astype(jnp.bfloat16)
        ph = lax.dot_general(oh_bf, vh, dn,
                             preferred_element_type=jnp.float32)
        return ph + lax.dot_general(oh_bf, vl, dn,
                                    preferred_element_type=jnp.float32)

    ps = segdot(x)
    psq = segdot(x * x)
    cnt = jnp.sum(onehot, axis=1, keepdims=True)

    @pl.when(i == 0)
    def _():
        psum_ref[...] = ps
        psq_ref[...] = psq
        cnt_ref[...] = cnt

    @pl.when(i > 0)
    def _():
        psum_ref[...] += ps
        psq_ref[...] += psq
        cnt_ref[...] += cnt


def _combine_body(psum_ref, psq_ref, phist_ref, tps_ref, tpsq_ref, tcnt_ref,
                  w_ref, b_ref, s_ref, a_ref, c_ref):
    sums = jnp.sum(psum_ref[...], axis=0) + tps_ref[...]
    sqs = jnp.sum(psq_ref[...], axis=0) + tpsq_ref[...]
    cnt = (jnp.sum(phist_ref[...], axis=0)[:, :1]
           + tcnt_ref[...])
    cnt = jnp.maximum(cnt, 1.0)
    mean = sums / cnt
    t = mean * s_ref[...]
    var_sum = sqs - 2.0 * t * sums + cnt * t * t
    var = jnp.maximum(var_sum, 0.0) / cnt
    rstd = lax.rsqrt(var + 1e-8)
    a = w_ref[...] * rstd
    a_ref[...] = a
    c_ref[...] = b_ref[...] - a * t


def _norm_process(ids_v, buf, obuf, av, cv, lo_eff, i):
    def gbody(g, c_):
        goff = g * 16
        idv = ids_v[pl.ds(i * UN + goff, 16)]
        s0 = idv[0] - lo_eff
        s15 = idv[15] - lo_eff

        def fast():
            for c in range(NC):
                sl = pl.ds(c * L, L)
                a_c = av[s0, sl]
                c_c = cv[s0, sl]
                for r in range(16):
                    row = goff + r
                    obuf[row, sl] = buf[row, sl] * a_c + c_c

        def slow():
            for r in range(16):
                s = idv[r] - lo_eff
                row = goff + r
                for c in range(NC):
                    sl = pl.ds(c * L, L)
                    obuf[row, sl] = buf[row, sl] * av[s, sl] + cv[s, sl]

        pl.when(s0 == s15)(fast)
        pl.when(s0 != s15)(slow)
        return c_

    lax.fori_loop(0, UN // 16, gbody, 0)


def _norm_body(x_hbm, ids_hbm, a_hbm, c_hbm, out_hbm,
               ids_v, xb0, xb1, ob0, ob1, av, cv,
               sem_i, sem0, sem1, semo0, semo1):
    wid = _wid()
    start_row = _start_row(wid)

    def run(n_u):
        n_rows = n_u * UN
        pltpu.async_copy(ids_hbm.at[pl.ds(start_row, n_rows)],
                         ids_v.at[pl.ds(0, n_rows)], sem_i)
        pltpu.make_async_copy(ids_hbm.at[pl.ds(start_row, n_rows)],
                              ids_v.at[pl.ds(0, n_rows)], sem_i).wait()

        lo = ids_v[pl.ds(0, 16)][0]
        hi = ids_v[pl.ds(n_rows - 16, 16)][15]
        lo_al = (lo // 8) * 8
        narrow = (hi - lo_al) < WSEG
        lo_eff = jnp.where(narrow, jnp.minimum(lo_al, G - WSEG), 0)

        def wide_load():
            pltpu.sync_copy(a_hbm, av)
            pltpu.sync_copy(c_hbm, cv)

        def narrow_load():
            pltpu.sync_copy(a_hbm.at[pl.ds(lo_eff, WSEG)],
                            av.at[pl.ds(0, WSEG)])
            pltpu.sync_copy(c_hbm.at[pl.ds(lo_eff, WSEG)],
                            cv.at[pl.ds(0, WSEG)])

        pl.when(narrow)(narrow_load)
        pl.when(jnp.logical_not(narrow))(wide_load)

        def issue_in(u, buf, sem):
            pltpu.async_copy(x_hbm.at[pl.ds(start_row + u * UN, UN), :],
                             buf, sem)

        def wait_in(u, buf, sem):
            pltpu.make_async_copy(x_hbm.at[pl.ds(start_row + u * UN, UN), :],
                                  buf, sem).wait()

        def issue_out(u, buf, sem):
            pltpu.async_copy(buf, out_hbm.at[pl.ds(start_row + u * UN, UN), :],
                             sem)

        def wait_out(u, buf, sem):
            pltpu.make_async_copy(buf,
                                  out_hbm.at[pl.ds(start_row + u * UN, UN), :],
                                  sem).wait()

        issue_in(0, xb0, sem0)
        issue_in(1, xb1, sem1)
        n_pairs = (n_u + 1) // 2

        def pbody(p, c_):
            i0 = 2 * p
            wait_in(i0, xb0, sem0)
            pl.when(p > 0)(lambda: wait_out(i0, ob0, semo0))
            _norm_process(ids_v, xb0, ob0, av, cv, lo_eff, i0)
            issue_out(i0, ob0, semo0)
            pl.when(i0 + 2 < n_u)(lambda: issue_in(i0 + 2, xb0, sem0))

            def second():
                wait_in(i0 + 1, xb1, sem1)
                pl.when(p > 0)(lambda: wait_out(i0 + 1, ob1, semo1))
                _norm_process(ids_v, xb1, ob1, av, cv, lo_eff, i0 + 1)
                issue_out(i0 + 1, ob1, semo1)
                pl.when(i0 + 3 < n_u)(lambda: issue_in(i0 + 3, xb1, sem1))

            pl.when(i0 + 1 < n_u)(second)
            return c_

        lax.fori_loop(0, n_pairs, pbody, 0)
        wait_out(0, ob0, semo0)
        wait_out(0, ob1, semo1)

    pl.when(wid < EXTRA)(lambda: run(BASE_U + 1))
    pl.when(wid >= EXTRA)(lambda: run(BASE_U))


def kernel(node_emb, segment_ids, weight, bias, scale):
    ids = segment_ids.astype(jnp.int32)
    mesh = plsc.VectorSubcoreMesh(core_axis_name="c", subcore_axis_name="s")

    stats = pl.kernel(
        _stats_body,
        mesh=mesh,
        out_type=(
            jax.ShapeDtypeStruct((NW, G, D), jnp.float32),
            jax.ShapeDtypeStruct((NW, G, D), jnp.float32),
            jax.ShapeDtypeStruct((NW, G, L), jnp.float32),
        ),
        scratch_types=[
            pltpu.VMEM((MAX_ROWS_S,), jnp.int32),
            pltpu.VMEM((U, D), jnp.float32),
            pltpu.VMEM((U, D), jnp.float32),
            pltpu.VMEM((G, D), jnp.float32),
            pltpu.VMEM((G, D), jnp.float32),
            pltpu.VMEM((G, L), jnp.float32),
            pltpu.SemaphoreType.DMA,
            pltpu.SemaphoreType.DMA,
            pltpu.SemaphoreType.DMA,
        ],
    )
    psum, psq, phist = stats(node_emb, ids)

    tps, tpsq, tcnt = pl.pallas_call(
        _tc_stats_body,
        grid=(N_TC // B_TC,),
        in_specs=[
            pl.BlockSpec((1, B_TC), lambda i: (0, i)),
            pl.BlockSpec((B_TC, D), lambda i: (i, 0)),
        ],
        out_specs=(
            pl.BlockSpec((G, D), lambda i: (0, 0)),
            pl.BlockSpec((G, D), lambda i: (0, 0)),
            pl.BlockSpec((G, 1), lambda i: (0, 0)),
        ),
        out_shape=(
            jax.ShapeDtypeStruct((G, D), jnp.float32),
            jax.ShapeDtypeStruct((G, D), jnp.float32),
            jax.ShapeDtypeStruct((G, 1), jnp.float32),
        ),
    )(ids.reshape(1, N), node_emb)

    a_tab, c_tab = pl.pallas_call(
        _combine_body,
        out_shape=(
            jax.ShapeDtypeStruct((G, D), jnp.float32),
            jax.ShapeDtypeStruct((G, D), jnp.float32),
        ),
    )(psum, psq, phist, tps, tpsq, tcnt,
      weight.reshape(1, D), bias.reshape(1, D), scale.reshape(1, D))

    norm = pl.kernel(
        _norm_body,
        mesh=mesh,
        out_type=jax.ShapeDtypeStruct((N, D), jnp.float32),
        scratch_types=[
            pltpu.VMEM((MAX_ROWS,), jnp.int32),
            pltpu.VMEM((UN, D), jnp.float32),
            pltpu.VMEM((UN, D), jnp.float32),
            pltpu.VMEM((UN, D), jnp.float32),
            pltpu.VMEM((UN, D), jnp.float32),
            pltpu.VMEM((G, D), jnp.float32),
            pltpu.VMEM((G, D), jnp.float32),
            pltpu.SemaphoreType.DMA,
            pltpu.SemaphoreType.DMA,
            pltpu.SemaphoreType.DMA,
            pltpu.SemaphoreType.DMA,
            pltpu.SemaphoreType.DMA,
        ],
    )
    return norm(node_emb, ids, a_tab, c_tab)


# speedup vs baseline: 7.4923x; 1.0154x over previous
import jax
import jax.numpy as jnp
from jax import lax
from jax.experimental import pallas as pl
from jax.experimental.pallas import tpu as pltpu
from jax.experimental.pallas import tpu_sc as plsc

N = 100000
D = 128
G = 256
L = 16
NC = D // L
NW = 32

U = 32
WSEG = 64

B_TC = 2048
N_TC = 57344
N_SC = N - N_TC

UN = 32
TOT_U = N // UN
BASE_U = TOT_U // NW
EXTRA = TOT_U - BASE_U * NW
MAX_ROWS = (BASE_U + 1) * UN

TOT_US = N_SC // U
BASE_US = TOT_US // NW
EXTRA_S = TOT_US - BASE_US * NW
MAX_ROWS_S = (BASE_US + 1) * U


def _wid():
    return lax.axis_index("s") * 2 + lax.axis_index("c")


def _start_row(wid):
    return (wid * BASE_U + jnp.minimum(wid, EXTRA)) * UN


def _start_row_s(wid):
    return N_TC + (wid * BASE_US + jnp.minimum(wid, EXTRA_S)) * U


def _tree_sum(vs):
    vs = list(vs)
    while len(vs) > 1:
        nxt = [vs[j] + vs[j + 1] for j in range(0, len(vs) - 1, 2)]
        if len(vs) % 2:
            nxt.append(vs[-1])
        vs = nxt
    return vs[0]


def _stats_process(ids_v, buf, acc, accsq, hist, i):
    ones = jnp.full((L,), 1.0, jnp.float32)
    sixteen = jnp.full((L,), 16.0, jnp.float32)

    def gbody(g, c_):
        goff = g * 16
        idv = ids_v[pl.ds(i * U + goff, 16)]
        s0 = idv[0]
        s15 = idv[15]

        def fast():
            for c in range(NC):
                sl = pl.ds(c * L, L)
                vals = [buf[goff + r, sl] for r in range(16)]
                plsc.addupdate(acc.at[s0, sl], _tree_sum(vals))
                plsc.addupdate(accsq.at[s0, sl],
                               _tree_sum([v * v for v in vals]))
            plsc.addupdate(hist.at[s0, :], sixteen)

        def slow():
            for r in range(16):
                s = idv[r]
                row = goff + r
                for c in range(NC):
                    sl = pl.ds(c * L, L)
                    xv = buf[row, sl]
                    plsc.addupdate(acc.at[s, sl], xv)
                    plsc.addupdate(accsq.at[s, sl], xv * xv)
                plsc.addupdate(hist.at[s, :], ones)

        pl.when(s0 == s15)(fast)
        pl.when(s0 != s15)(slow)
        return c_

    lax.fori_loop(0, U // 16, gbody, 0)


def _stats_body(x_hbm, ids_hbm, psum_hbm, psq_hbm, phist_hbm,
                ids_v, xb0, xb1, acc, accsq, hist, sem_i, sem0, sem1):
    wid = _wid()
    start_row = _start_row_s(wid)

    zero16 = jnp.zeros((L,), jnp.float32)

    def run(n_u):
        n_rows = n_u * U
        pltpu.async_copy(ids_hbm.at[pl.ds(start_row, n_rows)],
                         ids_v.at[pl.ds(0, n_rows)], sem_i)

        def zbody(g, c_):
            for c in range(NC):
                acc[g, pl.ds(c * L, L)] = zero16
                accsq[g, pl.ds(c * L, L)] = zero16
            hist[g, :] = zero16
            return c_

        lax.fori_loop(0, G, zbody, 0)

        pltpu.make_async_copy(ids_hbm.at[pl.ds(start_row, n_rows)],
                              ids_v.at[pl.ds(0, n_rows)], sem_i).wait()

        def issue(u, buf, sem):
            pltpu.async_copy(x_hbm.at[pl.ds(start_row + u * U, U), :],
                             buf, sem)

        def wait(u, buf, sem):
            pltpu.make_async_copy(x_hbm.at[pl.ds(start_row + u * U, U), :],
                                  buf, sem).wait()

        issue(0, xb0, sem0)
        issue(1, xb1, sem1)
        n_pairs = (n_u + 1) // 2

        def pbody(p, c_):
            i0 = 2 * p
            wait(i0, xb0, sem0)
            _stats_process(ids_v, xb0, acc, accsq, hist, i0)
            pl.when(i0 + 2 < n_u)(lambda: issue(i0 + 2, xb0, sem0))

            def second():
                wait(i0 + 1, xb1, sem1)
                _stats_process(ids_v, xb1, acc, accsq, hist, i0 + 1)
                pl.when(i0 + 3 < n_u)(lambda: issue(i0 + 3, xb1, sem1))

            pl.when(i0 + 1 < n_u)(second)
            return c_

        lax.fori_loop(0, n_pairs, pbody, 0)

    pl.when(wid < EXTRA_S)(lambda: run(BASE_US + 1))
    pl.when(wid >= EXTRA_S)(lambda: run(BASE_US))

    pltpu.sync_copy(acc, psum_hbm.at[wid])
    pltpu.sync_copy(accsq, psq_hbm.at[wid])
    pltpu.sync_copy(hist, phist_hbm.at[wid])


def _tc_stats_body(ids_ref, x_ref, psum_ref, psq_ref, cnt_ref):
    i = pl.program_id(0)
    x = x_ref[...]
    ids = ids_ref[...]
    seg = lax.broadcasted_iota(jnp.int32, (G, B_TC), 0)
    onehot = (ids == seg).astype(jnp.float32)
    oh_bf = onehot.astype(jnp.bfloat16)
    dn = (((1,), (0,)), ((), ()))

    def segdot(v):
        vh = v.astype(jnp.bfloat16)
        vl = (v - vh.astype(jnp.float32)).astype(jnp.bfloat16)
        ph = lax.dot_general(oh_bf, vh, dn,
                             preferred_element_type=jnp.float32)
        return ph + lax.dot_general(oh_bf, vl, dn,
                                    preferred_element_type=jnp.float32)

    ps = segdot(x)
    psq = segdot(x * x)
    cnt = jnp.sum(onehot, axis=1, keepdims=True)

    @pl.when(i == 0)
    def _():
        psum_ref[...] = ps
        psq_ref[...] = psq
        cnt_ref[...] = cnt

    @pl.when(i > 0)
    def _():
        psum_ref[...] += ps
        psq_ref[...] += psq
        cnt_ref[...] += cnt


def _combine_body(psum_ref, psq_ref, phist_ref, tps_ref, tpsq_ref, tcnt_ref,
                  w_ref, b_ref, s_ref, a_ref, c_ref):
    sums = jnp.sum(psum_ref[...], axis=0) + tps_ref[...]
    sqs = jnp.sum(psq_ref[...], axis=0) + tpsq_ref[...]
    cnt = (jnp.sum(phist_ref[...], axis=0)[:, :1]
           + tcnt_ref[...])
    cnt = jnp.maximum(cnt, 1.0)
    mean = sums / cnt
    t = mean * s_ref[...]
    var_sum = sqs - 2.0 * t * sums + cnt * t * t
    var = jnp.maximum(var_sum, 0.0) / cnt
    rstd = lax.rsqrt(var + 1e-8)
    a = w_ref[...] * rstd
    a_ref[...] = a
    c_ref[...] = b_ref[...] - a * t


def _norm_process(ids_v, buf, obuf, av, cv, lo_eff, i):
    def gbody(g, c_):
        goff = g * 16
        idv = ids_v[pl.ds(i * UN + goff, 16)]
        s0 = idv[0] - lo_eff
        s15 = idv[15] - lo_eff

        def fast():
            for c in range(NC):
                sl = pl.ds(c * L, L)
                a_c = av[s0, sl]
                c_c = cv[s0, sl]
                for r in range(16):
                    row = goff + r
                    obuf[row, sl] = buf[row, sl] * a_c + c_c

        def slow():
            for r in range(16):
                s = idv[r] - lo_eff
                row = goff + r
                for c in range(NC):
                    sl = pl.ds(c * L, L)
                    obuf[row, sl] = buf[row, sl] * av[s, sl] + cv[s, sl]

        pl.when(s0 == s15)(fast)
        pl.when(s0 != s15)(slow)
        return c_

    lax.fori_loop(0, UN // 16, gbody, 0)


def _norm_body(x_hbm, ids_hbm, a_hbm, c_hbm, out_hbm,
               ids_v, xb0, xb1, ob0, ob1, av, cv,
               sem_i, sem0, sem1, semo0, semo1):
    wid = _wid()
    start_row = _start_row(wid)

    def run(n_u):
        n_rows = n_u * UN
        pltpu.async_copy(ids_hbm.at[pl.ds(start_row, n_rows)],
                         ids_v.at[pl.ds(0, n_rows)], sem_i)
        pltpu.make_async_copy(ids_hbm.at[pl.ds(start_row, n_rows)],
                              ids_v.at[pl.ds(0, n_rows)], sem_i).wait()

        lo = ids_v[pl.ds(0, 16)][0]
        hi = ids_v[pl.ds(n_rows - 16, 16)][15]
        lo_al = (lo // 8) * 8
        narrow = (hi - lo_al) < WSEG
        lo_eff = jnp.where(narrow, jnp.minimum(lo_al, G - WSEG), 0)

        def wide_load():
            pltpu.sync_copy(a_hbm, av)
            pltpu.sync_copy(c_hbm, cv)

        def narrow_load():
            pltpu.sync_copy(a_hbm.at[pl.ds(lo_eff, WSEG)],
                            av.at[pl.ds(0, WSEG)])
            pltpu.sync_copy(c_hbm.at[pl.ds(lo_eff, WSEG)],
                            cv.at[pl.ds(0, WSEG)])

        pl.when(narrow)(narrow_load)
        pl.when(jnp.logical_not(narrow))(wide_load)

        def issue_in(u, buf, sem):
            pltpu.async_copy(x_hbm.at[pl.ds(start_row + u * UN, UN), :],
                             buf, sem)

        def wait_in(u, buf, sem):
            pltpu.make_async_copy(x_hbm.at[pl.ds(start_row + u * UN, UN), :],
                                  buf, sem).wait()

        def issue_out(u, buf, sem):
            pltpu.async_copy(buf, out_hbm.at[pl.ds(start_row + u * UN, UN), :],
                             sem)

        def wait_out(u, buf, sem):
            pltpu.make_async_copy(buf,
                                  out_hbm.at[pl.ds(start_row + u * UN, UN), :],
                                  sem).wait()

        issue_in(0, xb0, sem0)
        issue_in(1, xb1, sem1)
        n_pairs = (n_u + 1) // 2

        def pbody(p, c_):
            i0 = 2 * p
            wait_in(i0, xb0, sem0)
            pl.when(p > 0)(lambda: wait_out(i0, ob0, semo0))
            _norm_process(ids_v, xb0, ob0, av, cv, lo_eff, i0)
            issue_out(i0, ob0, semo0)
            pl.when(i0 + 2 < n_u)(lambda: issue_in(i0 + 2, xb0, sem0))

            def second():
                wait_in(i0 + 1, xb1, sem1)
                pl.when(p > 0)(lambda: wait_out(i0 + 1, ob1, semo1))
                _norm_process(ids_v, xb1, ob1, av, cv, lo_eff, i0 + 1)
                issue_out(i0 + 1, ob1, semo1)
                pl.when(i0 + 3 < n_u)(lambda: issue_in(i0 + 3, xb1, sem1))

            pl.when(i0 + 1 < n_u)(second)
            return c_

        lax.fori_loop(0, n_pairs, pbody, 0)
        wait_out(0, ob0, semo0)
        wait_out(0, ob1, semo1)

    pl.when(wid < EXTRA)(lambda: run(BASE_U + 1))
    pl.when(wid >= EXTRA)(lambda: run(BASE_U))


def kernel(node_emb, segment_ids, weight, bias, scale):
    ids = segment_ids.astype(jnp.int32)
    mesh = plsc.VectorSubcoreMesh(core_axis_name="c", subcore_axis_name="s")

    stats = pl.kernel(
        _stats_body,
        mesh=mesh,
        out_type=(
            jax.ShapeDtypeStruct((NW, G, D), jnp.float32),
            jax.ShapeDtypeStruct((NW, G, D), jnp.float32),
            jax.ShapeDtypeStruct((NW, G, L), jnp.float32),
        ),
        scratch_types=[
            pltpu.VMEM((MAX_ROWS_S,), jnp.int32),
            pltpu.VMEM((U, D), jnp.float32),
            pltpu.VMEM((U, D), jnp.float32),
            pltpu.VMEM((G, D), jnp.float32),
            pltpu.VMEM((G, D), jnp.float32),
            pltpu.VMEM((G, L), jnp.float32),
            pltpu.SemaphoreType.DMA,
            pltpu.SemaphoreType.DMA,
            pltpu.SemaphoreType.DMA,
        ],
    )
    psum, psq, phist = stats(node_emb, ids)

    tps, tpsq, tcnt = pl.pallas_call(
        _tc_stats_body,
        grid=(N_TC // B_TC,),
        in_specs=[
            pl.BlockSpec((1, B_TC), lambda i: (0, i)),
            pl.BlockSpec((B_TC, D), lambda i: (i, 0)),
        ],
        out_specs=(
            pl.BlockSpec((G, D), lambda i: (0, 0)),
            pl.BlockSpec((G, D), lambda i: (0, 0)),
            pl.BlockSpec((G, 1), lambda i: (0, 0)),
        ),
        out_shape=(
            jax.ShapeDtypeStruct((G, D), jnp.float32),
            jax.ShapeDtypeStruct((G, D), jnp.float32),
            jax.ShapeDtypeStruct((G, 1), jnp.float32),
        ),
    )(ids.reshape(1, N), node_emb)

    a_tab, c_tab = pl.pallas_call(
        _combine_body,
        out_shape=(
            jax.ShapeDtypeStruct((G, D), jnp.float32),
            jax.ShapeDtypeStruct((G, D), jnp.float32),
        ),
    )(psum, psq, phist, tps, tpsq, tcnt,
      weight.reshape(1, D), bias.reshape(1, D), scale.reshape(1, D))

    norm = pl.kernel(
        _norm_body,
        mesh=mesh,
        out_type=jax.ShapeDtypeStruct((N, D), jnp.float32),
        scratch_types=[
            pltpu.VMEM((MAX_ROWS,), jnp.int32),
            pltpu.VMEM((UN, D), jnp.float32),
            pltpu.VMEM((UN, D), jnp.float32),
            pltpu.VMEM((UN, D), jnp.float32),
            pltpu.VMEM((UN, D), jnp.float32),
            pltpu.VMEM((G, D), jnp.float32),
            pltpu.VMEM((G, D), jnp.float32),
            pltpu.SemaphoreType.DMA,
            pltpu.SemaphoreType.DMA,
            pltpu.SemaphoreType.DMA,
            pltpu.SemaphoreType.DMA,
            pltpu.SemaphoreType.DMA,
        ],
    )
    return norm(node_emb, ids, a_tab, c_tab)
